# Initial kernel scaffold; baseline (speedup 1.0000x reference)
import functools

import jax
import jax.numpy as jnp
from jax import lax
from jax.experimental import pallas as pl
from jax.experimental.pallas import tpu as pltpu
from jax.experimental.pallas import tpu_sc as plsc

N = 10000
NP = 10240
D = 128
E = 320000
NC = 2
NS = 16
NW = NC * NS
L = 16
EPW = E // NW
K = 80
CH = EPW // K
RPS = NP // NS


_TCB = 2560


def _tc_first_body(x_ref, w_ref, as_ref, ad_ref, h_ref, s_ref, d_ref):
    h = jnp.dot(x_ref[...], w_ref[...], preferred_element_type=jnp.float32)
    h_ref[...] = h
    s_ref[...] = jnp.sum(h * as_ref[...][None, :], axis=1)
    d_ref[...] = jnp.sum(h * ad_ref[...][None, :], axis=1)


def _tc_first(x_p, W, a_s, a_d):
    return pl.pallas_call(
        _tc_first_body,
        grid=(NP // _TCB,),
        in_specs=[
            pl.BlockSpec((_TCB, D), lambda i: (i, 0)),
            pl.BlockSpec((D, D), lambda i: (0, 0)),
            pl.BlockSpec((D,), lambda i: (0,)),
            pl.BlockSpec((D,), lambda i: (0,)),
        ],
        out_specs=[
            pl.BlockSpec((_TCB, D), lambda i: (i, 0)),
            pl.BlockSpec((_TCB,), lambda i: (i,)),
            pl.BlockSpec((_TCB,), lambda i: (i,)),
        ],
        out_shape=[
            jax.ShapeDtypeStruct((NP, D), jnp.float32),
            jax.ShapeDtypeStruct((NP,), jnp.float32),
            jax.ShapeDtypeStruct((NP,), jnp.float32),
        ],
    )(x_p, W, a_s, a_d)


def _normalized(acc_ref, den_ref, b_ref):
    a = jnp.sum(acc_ref[...], axis=0)
    dn = jnp.sum(den_ref[...], axis=1, keepdims=True)
    o = a / jnp.maximum(dn, 1e-16) + b_ref[...][None, :]
    return jnp.maximum(o, 0.0)


def _tc_mid_body(acc_ref, den_ref, b_ref, w_ref, as_ref, ad_ref,
                 h_ref, s_ref, d_ref):
    o = _normalized(acc_ref, den_ref, b_ref)
    h = jnp.dot(o, w_ref[...], preferred_element_type=jnp.float32)
    h_ref[...] = h
    s_ref[...] = jnp.sum(h * as_ref[...][None, :], axis=1)
    d_ref[...] = jnp.sum(h * ad_ref[...][None, :], axis=1)


def _tc_mid(acc, den_t, b, W, a_s, a_d):
    return pl.pallas_call(
        _tc_mid_body,
        grid=(NP // _TCB,),
        in_specs=[
            pl.BlockSpec((2, _TCB, D), lambda i: (0, i, 0)),
            pl.BlockSpec((_TCB, 2), lambda i: (i, 0)),
            pl.BlockSpec((D,), lambda i: (0,)),
            pl.BlockSpec((D, D), lambda i: (0, 0)),
            pl.BlockSpec((D,), lambda i: (0,)),
            pl.BlockSpec((D,), lambda i: (0,)),
        ],
        out_specs=[
            pl.BlockSpec((_TCB, D), lambda i: (i, 0)),
            pl.BlockSpec((_TCB,), lambda i: (i,)),
            pl.BlockSpec((_TCB,), lambda i: (i,)),
        ],
        out_shape=[
            jax.ShapeDtypeStruct((NP, D), jnp.float32),
            jax.ShapeDtypeStruct((NP,), jnp.float32),
            jax.ShapeDtypeStruct((NP,), jnp.float32),
        ],
    )(acc, den_t, b, W, a_s, a_d)


def _tc_last_body(acc_ref, den_ref, b_ref, wf_ref, bf_ref, y_ref):
    o = _normalized(acc_ref, den_ref, b_ref)
    y_ref[...] = jnp.sum(o * wf_ref[...][None, :], axis=1) + bf_ref[0]


def _tc_last(acc, den_t, b, wf, bf):
    return pl.pallas_call(
        _tc_last_body,
        grid=(NP // _TCB,),
        in_specs=[
            pl.BlockSpec((2, _TCB, D), lambda i: (0, i, 0)),
            pl.BlockSpec((_TCB, 2), lambda i: (i, 0)),
            pl.BlockSpec((D,), lambda i: (0,)),
            pl.BlockSpec((D,), lambda i: (0,)),
            pl.BlockSpec(memory_space=pltpu.SMEM),
        ],
        out_specs=pl.BlockSpec((_TCB,), lambda i: (i,)),
        out_shape=jax.ShapeDtypeStruct((NP,), jnp.float32),
    )(acc, den_t, b, wf, bf)


def _sc_edge_body(h_hbm, s_hbm, d_hbm, src_hbm, dst_hbm,
                  acc_out, den_out,
                  src_v, dst_v, rows_v, val_v, asrc_v, adst_v, den_v,
                  dred_v, acc_sh, den_stage, sem):
    cid = lax.axis_index("c")
    sid = lax.axis_index("s")
    wid = sid * NC + cid
    base = wid * EPW

    zero16 = jnp.zeros((L,), jnp.float32)

    def zden(i, carry):
        den_v[pl.ds(i * L, L)] = zero16
        return carry
    lax.fori_loop(0, NP // L, zden, 0)

    def zrow(j, carry):
        for c8 in range(D // L):
            rows_v[j, pl.ds(c8 * L, L)] = zero16
        return carry
    lax.fori_loop(0, K, zrow, 0)
    for k in range(RPS // K):
        pltpu.sync_copy(rows_v, acc_sh.at[pl.ds(sid * RPS + k * K, K)])

    pltpu.sync_copy(s_hbm, asrc_v)
    pltpu.sync_copy(d_hbm, adst_v)
    plsc.subcore_barrier()

    def chunk(c, carry):
        off = pl.multiple_of(base + c * K, 8)
        pltpu.sync_copy(src_hbm.at[pl.ds(off, K)], src_v)
        pltpu.sync_copy(dst_hbm.at[pl.ds(off, K)], dst_v)
        cp = pltpu.async_copy(h_hbm.at[src_v], rows_v, sem)

        def vals(j, carry2):
            si = src_v[pl.ds(j * L, L)]
            di = dst_v[pl.ds(j * L, L)]
            e = plsc.load_gather(asrc_v, [si]) + plsc.load_gather(adst_v, [di])
            e = jnp.where(e >= 0.0, e, 0.2 * e)
            v = jnp.exp(e)
            val_v[pl.ds(j * L, L)] = v
            plsc.addupdate_scatter(den_v, [di], v)
            return carry2
        lax.fori_loop(0, K // L, vals, 0)

        cp.wait()

        def scale(j, carry2):
            v = val_v[j]
            for c8 in range(D // L):
                rows_v[j, pl.ds(c8 * L, L)] = rows_v[j, pl.ds(c8 * L, L)] * v
            return carry2
        lax.fori_loop(0, K, scale, 0)

        pltpu.sync_copy(rows_v, acc_sh.at[dst_v], add=True)
        return carry
    lax.fori_loop(0, CH, chunk, 0)

    pltpu.sync_copy(den_v, den_stage.at[sid])
    plsc.subcore_barrier()
    pltpu.sync_copy(den_stage.at[:, pl.ds(sid * RPS, RPS)], dred_v)

    def dsum(j, carry):
        t = zero16
        for r in range(NS):
            t = t + dred_v[r, pl.ds(j * L, L)]
        den_v[pl.ds(sid * RPS + j * L, L)] = t
        return carry
    lax.fori_loop(0, RPS // L, dsum, 0)

    pltpu.sync_copy(den_v.at[pl.ds(sid * RPS, RPS)],
                    den_out.at[cid, pl.ds(sid * RPS, RPS)])
    pltpu.sync_copy(acc_sh.at[pl.ds(sid * RPS, RPS)],
                    acc_out.at[cid, pl.ds(sid * RPS, RPS)])


def _sc_edge(h, s, d, src, dst):
    mesh = plsc.VectorSubcoreMesh(core_axis_name="c", subcore_axis_name="s")
    f = pl.kernel(
        _sc_edge_body,
        out_type=[
            jax.ShapeDtypeStruct((2, NP, D), jnp.float32),
            jax.ShapeDtypeStruct((2, NP), jnp.float32),
        ],
        mesh=mesh,
        scratch_types=[
            pltpu.VMEM((K,), jnp.int32),
            pltpu.VMEM((K,), jnp.int32),
            pltpu.VMEM((K, D), jnp.float32),
            pltpu.VMEM((K,), jnp.float32),
            pltpu.VMEM((NP,), jnp.float32),
            pltpu.VMEM((NP,), jnp.float32),
            pltpu.VMEM((NP,), jnp.float32),
            pltpu.VMEM((NS, RPS), jnp.float32),
            pltpu.VMEM_SHARED((NP, D), jnp.float32),
            pltpu.VMEM_SHARED((NS, NP), jnp.float32),
            pltpu.SemaphoreType.DMA,
        ],
    )
    return f(h, s, d, src, dst)


def kernel(x, edge_indices, W1, a_src1, a_dst1, b1,
           W2, a_src2, a_dst2, b2, Wf, bf):
    xs = jnp.squeeze(x, axis=0)
    x_p = jnp.pad(xs, ((0, NP - N), (0, 0)))
    ei = jnp.squeeze(edge_indices, axis=0)
    src = ei[:, 0].astype(jnp.int32)
    dst = ei[:, 1].astype(jnp.int32)

    h1, s1, d1 = _tc_first(x_p, W1, a_src1.reshape(D), a_dst1.reshape(D))
    acc1, den1 = _sc_edge(h1, s1, d1, src, dst)
    h2, s2, d2 = _tc_mid(acc1, den1.T, b1, W2,
                         a_src2.reshape(D), a_dst2.reshape(D))
    acc2, den2 = _sc_edge(h2, s2, d2, src, dst)
    y = _tc_last(acc2, den2.T, b2, Wf.reshape(D), bf)
    return y[:N]


# speedup vs baseline: 24.8497x; 24.8497x over previous
import functools

import jax
import jax.numpy as jnp
from jax import lax
from jax.experimental import pallas as pl
from jax.experimental.pallas import tpu as pltpu
from jax.experimental.pallas import tpu_sc as plsc

N = 10000
NP = 10240
D = 128
E = 320000
NC = 2
NS = 16
NW = NC * NS
L = 16
EPW = E // NW
K = 80
CH = EPW // K
RPS = NP // NS


_TCB = 2048


def _tc_first_body(x_ref, w_ref, as_ref, ad_ref, h_ref, s_ref, d_ref):
    h = jnp.dot(x_ref[...], w_ref[...], preferred_element_type=jnp.float32)
    h_ref[...] = h
    s_ref[...] = jnp.sum(h * as_ref[...][None, :], axis=1)
    d_ref[...] = jnp.sum(h * ad_ref[...][None, :], axis=1)


def _tc_first(x_p, W, a_s, a_d):
    return pl.pallas_call(
        _tc_first_body,
        grid=(NP // _TCB,),
        in_specs=[
            pl.BlockSpec((_TCB, D), lambda i: (i, 0)),
            pl.BlockSpec((D, D), lambda i: (0, 0)),
            pl.BlockSpec((D,), lambda i: (0,)),
            pl.BlockSpec((D,), lambda i: (0,)),
        ],
        out_specs=[
            pl.BlockSpec((_TCB, D), lambda i: (i, 0)),
            pl.BlockSpec((_TCB,), lambda i: (i,)),
            pl.BlockSpec((_TCB,), lambda i: (i,)),
        ],
        out_shape=[
            jax.ShapeDtypeStruct((NP, D), jnp.float32),
            jax.ShapeDtypeStruct((NP,), jnp.float32),
            jax.ShapeDtypeStruct((NP,), jnp.float32),
        ],
    )(x_p, W, a_s, a_d)


def _normalized(acc_ref, den_ref, b_ref):
    a = jnp.sum(acc_ref[...], axis=0)
    dn = jnp.sum(den_ref[...], axis=1, keepdims=True)
    o = a / jnp.maximum(dn, 1e-16) + b_ref[...][None, :]
    return jnp.maximum(o, 0.0)


def _tc_mid_body(acc_ref, den_ref, b_ref, w_ref, as_ref, ad_ref,
                 h_ref, s_ref, d_ref):
    o = _normalized(acc_ref, den_ref, b_ref)
    h = jnp.dot(o, w_ref[...], preferred_element_type=jnp.float32)
    h_ref[...] = h
    s_ref[...] = jnp.sum(h * as_ref[...][None, :], axis=1)
    d_ref[...] = jnp.sum(h * ad_ref[...][None, :], axis=1)


def _tc_mid(acc, den_t, b, W, a_s, a_d):
    return pl.pallas_call(
        _tc_mid_body,
        grid=(NP // _TCB,),
        in_specs=[
            pl.BlockSpec((2, _TCB, D), lambda i: (0, i, 0)),
            pl.BlockSpec((_TCB, NW), lambda i: (i, 0)),
            pl.BlockSpec((D,), lambda i: (0,)),
            pl.BlockSpec((D, D), lambda i: (0, 0)),
            pl.BlockSpec((D,), lambda i: (0,)),
            pl.BlockSpec((D,), lambda i: (0,)),
        ],
        out_specs=[
            pl.BlockSpec((_TCB, D), lambda i: (i, 0)),
            pl.BlockSpec((_TCB,), lambda i: (i,)),
            pl.BlockSpec((_TCB,), lambda i: (i,)),
        ],
        out_shape=[
            jax.ShapeDtypeStruct((NP, D), jnp.float32),
            jax.ShapeDtypeStruct((NP,), jnp.float32),
            jax.ShapeDtypeStruct((NP,), jnp.float32),
        ],
    )(acc, den_t, b, W, a_s, a_d)


def _tc_last_body(acc_ref, den_ref, b_ref, wf_ref, bf_ref, y_ref):
    o = _normalized(acc_ref, den_ref, b_ref)
    y_ref[...] = jnp.sum(o * wf_ref[...][None, :], axis=1) + bf_ref[0]


def _tc_last(acc, den_t, b, wf, bf):
    return pl.pallas_call(
        _tc_last_body,
        grid=(NP // _TCB,),
        in_specs=[
            pl.BlockSpec((2, _TCB, D), lambda i: (0, i, 0)),
            pl.BlockSpec((_TCB, NW), lambda i: (i, 0)),
            pl.BlockSpec((D,), lambda i: (0,)),
            pl.BlockSpec((D,), lambda i: (0,)),
            pl.BlockSpec(memory_space=pltpu.SMEM),
        ],
        out_specs=pl.BlockSpec((_TCB,), lambda i: (i,)),
        out_shape=jax.ShapeDtypeStruct((NP,), jnp.float32),
    )(acc, den_t, b, wf, bf)


def _sc_edge_body(h_hbm, s_hbm, d_hbm, src_hbm, dst_hbm,
                  acc_out, den_out,
                  src_v, dst_v, rows_v, val_v, asrc_v, adst_v, den_v,
                  acc_sh, sem):
    cid = lax.axis_index("c")
    sid = lax.axis_index("s")
    wid = sid * NC + cid
    base = wid * EPW

    zero16 = jnp.zeros((L,), jnp.float32)

    def zden(i, carry):
        den_v[pl.ds(i * L, L)] = zero16
        return carry
    lax.fori_loop(0, NP // L, zden, 0)

    def zrow(j, carry):
        for c8 in range(D // L):
            rows_v[j, pl.ds(c8 * L, L)] = zero16
        return carry
    lax.fori_loop(0, K, zrow, 0)
    for k in range(RPS // K):
        pltpu.sync_copy(rows_v, acc_sh.at[pl.ds(sid * RPS + k * K, K)])

    pltpu.sync_copy(s_hbm, asrc_v)
    pltpu.sync_copy(d_hbm, adst_v)
    plsc.subcore_barrier()

    def chunk(c, carry):
        off = pl.multiple_of(base + c * K, 8)
        pltpu.sync_copy(src_hbm.at[pl.ds(off, K)], src_v)
        pltpu.sync_copy(dst_hbm.at[pl.ds(off, K)], dst_v)
        cp = pltpu.async_copy(h_hbm.at[src_v], rows_v, sem)

        def vals(j, carry2):
            si = src_v[pl.ds(j * L, L)]
            di = dst_v[pl.ds(j * L, L)]
            e = plsc.load_gather(asrc_v, [si]) + plsc.load_gather(adst_v, [di])
            e = jnp.where(e >= 0.0, e, 0.2 * e)
            v = jnp.exp(e)
            val_v[pl.ds(j * L, L)] = v
            plsc.addupdate_scatter(den_v, [di], v)
            return carry2
        lax.fori_loop(0, K // L, vals, 0)

        cp.wait()

        def scale(g, carry2):
            vv = val_v[pl.ds(g * L, L)]
            for i in range(L):
                v = vv[i]
                j = g * L + i
                for c8 in range(D // L):
                    rows_v[j, pl.ds(c8 * L, L)] = (
                        rows_v[j, pl.ds(c8 * L, L)] * v)
            return carry2
        lax.fori_loop(0, K // L, scale, 0)

        pltpu.sync_copy(rows_v, acc_sh.at[dst_v], add=True)
        return carry
    lax.fori_loop(0, CH, chunk, 0)

    pltpu.sync_copy(den_v, den_out.at[cid, sid])
    plsc.subcore_barrier()
    pltpu.sync_copy(acc_sh.at[pl.ds(sid * RPS, RPS)],
                    acc_out.at[cid, pl.ds(sid * RPS, RPS)])


def _sc_edge(h, s, d, src, dst):
    mesh = plsc.VectorSubcoreMesh(core_axis_name="c", subcore_axis_name="s")
    f = pl.kernel(
        _sc_edge_body,
        out_type=[
            jax.ShapeDtypeStruct((2, NP, D), jnp.float32),
            jax.ShapeDtypeStruct((2, NS, NP), jnp.float32),
        ],
        mesh=mesh,
        scratch_types=[
            pltpu.VMEM((K,), jnp.int32),
            pltpu.VMEM((K,), jnp.int32),
            pltpu.VMEM((K, D), jnp.float32),
            pltpu.VMEM((K,), jnp.float32),
            pltpu.VMEM((NP,), jnp.float32),
            pltpu.VMEM((NP,), jnp.float32),
            pltpu.VMEM((NP,), jnp.float32),
            pltpu.VMEM_SHARED((NP, D), jnp.float32),
            pltpu.SemaphoreType.DMA,
        ],
        compiler_params=pltpu.CompilerParams(needs_layout_passes=False),
    )
    return f(h, s, d, src, dst)


def kernel(x, edge_indices, W1, a_src1, a_dst1, b1,
           W2, a_src2, a_dst2, b2, Wf, bf):
    xs = jnp.squeeze(x, axis=0)
    x_p = jnp.pad(xs, ((0, NP - N), (0, 0)))
    ei = jnp.squeeze(edge_indices, axis=0)
    src = ei[:, 0].astype(jnp.int32)
    dst = ei[:, 1].astype(jnp.int32)

    h1, s1, d1 = _tc_first(x_p, W1, a_src1.reshape(D), a_dst1.reshape(D))
    acc1, den1 = _sc_edge(h1, s1, d1, src, dst)
    h2, s2, d2 = _tc_mid(acc1, den1.reshape(NW, NP).T, b1, W2,
                         a_src2.reshape(D), a_dst2.reshape(D))
    acc2, den2 = _sc_edge(h2, s2, d2, src, dst)
    y = _tc_last(acc2, den2.reshape(NW, NP).T, b2, Wf.reshape(D), bf)
    return y[:N]
